# baseline (speedup 1.0000x reference)
import functools

import jax
import jax.numpy as jnp
from jax import lax
from jax.experimental import pallas as pl
from jax.experimental.pallas import tpu as pltpu
from jax.experimental.pallas import tpu_sc as plsc

_V = 1000000
_D = 32
_B = 16384
_L = 16
_CH = 128


@functools.lru_cache(maxsize=1)
def _build_sc_dots():
    info = plsc.get_sparse_core_info()
    NC, NS = info.num_cores, info.num_subcores
    NW = NC * NS
    bpw = _B // NW
    nch = bpw // _CH
    groups = bpw // _L

    mesh = plsc.VectorSubcoreMesh(core_axis_name="c", subcore_axis_name="s")

    @functools.partial(
        pl.kernel,
        out_type=[
            jax.ShapeDtypeStruct((_B,), jnp.float32),
            jax.ShapeDtypeStruct((_B,), jnp.float32),
        ],
        mesh=mesh,
        scratch_types=[
            pltpu.VMEM((nch, _CH), jnp.int32),
            pltpu.VMEM((nch, _CH), jnp.int32),
            pltpu.VMEM((nch, _CH), jnp.int32),
            pltpu.VMEM((bpw, _D), jnp.float32),
            pltpu.VMEM((bpw, _D), jnp.float32),
            pltpu.VMEM((bpw, _D), jnp.float32),
            pltpu.VMEM((bpw,), jnp.float32),
            pltpu.VMEM((bpw,), jnp.float32),
            pltpu.SemaphoreType.DMA,
        ],
        compiler_params=pltpu.CompilerParams(
            use_tc_tiling_on_sc=False, needs_layout_passes=False
        ),
    )
    def dots(iw_hbm, ow_hbm, nw_hbm, ei_hbm, eo_hbm, do_hbm, dn_hbm,
             iw_v, ow_v, nw_v, iv_v, ov_v, nv_v, do_v, dn_v, sem):
        wid = lax.axis_index("s") * NC + lax.axis_index("c")
        pltpu.sync_copy(iw_hbm.at[wid], iw_v)
        pltpu.sync_copy(ow_hbm.at[wid], ow_v)
        pltpu.sync_copy(nw_hbm.at[wid], nw_v)
        cps = []
        for c in range(nch):
            dst = pl.ds(c * _CH, _CH)
            cps.append(pltpu.async_copy(ei_hbm.at[iw_v.at[c]], iv_v.at[dst], sem))
            cps.append(pltpu.async_copy(eo_hbm.at[ow_v.at[c]], ov_v.at[dst], sem))
            cps.append(pltpu.async_copy(eo_hbm.at[nw_v.at[c]], nv_v.at[dst], sem))
        for cp in cps:
            cp.wait()

        def group_body(g, carry):
            rows = g * _L + lax.iota(jnp.int32, _L)
            acc_o = jnp.zeros((_L,), jnp.float32)
            acc_n = jnp.zeros((_L,), jnp.float32)
            for d in range(_D):
                dd = jnp.full((_L,), d, jnp.int32)
                iv = plsc.load_gather(iv_v, [rows, dd])
                ov = plsc.load_gather(ov_v, [rows, dd])
                nv = plsc.load_gather(nv_v, [rows, dd])
                acc_o = acc_o + iv * ov
                acc_n = acc_n + iv * nv
            do_v[pl.ds(g * _L, _L)] = acc_o
            dn_v[pl.ds(g * _L, _L)] = acc_n
            return carry

        lax.fori_loop(0, groups, group_body, 0)
        pltpu.sync_copy(do_v, do_hbm.at[pl.ds(wid * bpw, bpw)])
        pltpu.sync_copy(dn_v, dn_hbm.at[pl.ds(wid * bpw, bpw)])

    return dots, NW, nch


def _loss_body(do_ref, dn_ref, out_ref):
    x = do_ref[...]
    y = -dn_ref[...]
    ls = jnp.minimum(x, 0.0) - jnp.log1p(jnp.exp(-jnp.abs(x)))
    ls = ls + jnp.minimum(y, 0.0) - jnp.log1p(jnp.exp(-jnp.abs(y)))
    out_ref[0, 0] = -jnp.sum(ls) / _B


@functools.lru_cache(maxsize=1)
def _build_loss():
    return pl.pallas_call(
        _loss_body,
        out_shape=jax.ShapeDtypeStruct((1, 1), jnp.float32),
        out_specs=pl.BlockSpec(memory_space=pltpu.SMEM),
    )


@jax.jit
def kernel(iword, owords, nwords, emb_i, emb_o):
    dots, NW, nch = _build_sc_dots()
    iw = iword.astype(jnp.int32).reshape(NW, nch, _CH)
    ow = owords.astype(jnp.int32).reshape(NW, nch, _CH)
    nw = nwords.astype(jnp.int32).reshape(NW, nch, _CH)
    do, dn = dots(iw, ow, nw, emb_i, emb_o)
    loss = _build_loss()(do.reshape(128, 128), dn.reshape(128, 128))
    return loss[0, 0]
